# baseline (speedup 1.0000x reference)
import functools

import numpy as np
import jax
import jax.numpy as jnp
from jax import lax
from jax.experimental import pallas as pl
from jax.experimental.pallas import tpu as pltpu
from jax.experimental.pallas import tpu_sc as plsc

N = 10000
G = 16
K = 16
H = 128
NP = 10240
RB = 128
CB = 512
NBLK = NP // RB
BIG = 1e10
HUGE = 1e30
NW = 32
NODES_W = NP // NW
CHN = 8
EDG_W = NODES_W * K

_rs17 = np.float32(1.0) / np.sqrt(np.float32(17.0))
C17 = float(np.float32(_rs17 * _rs17))
_HP = lax.Precision.HIGHEST


def _knn_body(clo_ref, ccnt_ref, posr_ref, posc_ref, sqr_ref, sqc_ref,
              br_ref, bc_ref, topi_ref):
    i = pl.program_id(0)
    r0 = i * RB
    rbatch = br_ref[pl.ds(r0, RB), :]
    riota = r0 + lax.broadcasted_iota(jnp.int32, (RB, 1), 0)
    prow = posr_ref[pl.ds(r0, RB), :]
    sqr = sqr_ref[pl.ds(r0, RB), :]

    topv0 = jnp.full((RB, K), HUGE, jnp.float32)
    topi0 = jnp.zeros((RB, K), jnp.int32)
    cblk0 = clo_ref[i]

    def chunk(t, carry):
        topv, topi = carry
        c = (cblk0 + t) * CB
        dotv = jnp.dot(prow, posc_ref[:, pl.ds(c, CB)],
                       preferred_element_type=jnp.float32)
        d2 = (sqr + sqc_ref[0:1, pl.ds(c, CB)]) - 2.0 * dotv
        cbatch = bc_ref[0:1, pl.ds(c, CB)]
        ciota = c + lax.broadcasted_iota(jnp.int32, (1, CB), 1)
        masked = (cbatch != rbatch) | (ciota == riota)
        d2 = jnp.where(masked, jnp.float32(BIG), d2)

        A = jnp.concatenate([topv, d2], axis=1)
        I = jnp.concatenate([topi, jnp.broadcast_to(ciota, (RB, CB))], axis=1)
        vs, js = [], []
        for _ in range(K):
            m = jnp.min(A, axis=1, keepdims=True)
            sel = A == m
            j = jnp.min(jnp.where(sel, I, jnp.int32(2147483647)),
                        axis=1, keepdims=True)
            vs.append(m)
            js.append(j)
            A = jnp.where(sel & (I == j), jnp.float32(HUGE), A)
        return (jnp.concatenate(vs, axis=1), jnp.concatenate(js, axis=1))

    topv, topi = lax.fori_loop(0, ccnt_ref[i], chunk, (topv0, topi0))
    topi_ref[...] = topi


def _knn_topk(pos8, posT8, sq_r, sq_c, batch_r, batch_c, clo, ccnt):
    grid_spec = pltpu.PrefetchScalarGridSpec(
        num_scalar_prefetch=2,
        grid=(NBLK,),
        in_specs=[
            pl.BlockSpec((NP, 8), lambda i, *_: (0, 0)),
            pl.BlockSpec((8, NP), lambda i, *_: (0, 0)),
            pl.BlockSpec((NP, 1), lambda i, *_: (0, 0)),
            pl.BlockSpec((1, NP), lambda i, *_: (0, 0)),
            pl.BlockSpec((NP, 1), lambda i, *_: (0, 0)),
            pl.BlockSpec((1, NP), lambda i, *_: (0, 0)),
        ],
        out_specs=pl.BlockSpec((RB, K), lambda i, *_: (i, 0)),
    )
    return pl.pallas_call(
        _knn_body,
        grid_spec=grid_spec,
        out_shape=jax.ShapeDtypeStruct((NP, K), jnp.int32),
    )(clo, ccnt, pos8, posT8, sq_r, sq_c, batch_r, batch_c)


def _lin_body(x_ref, w_ref, out_ref):
    out_ref[...] = jnp.dot(x_ref[...], w_ref[...],
                           preferred_element_type=jnp.float32, precision=_HP)


def _lin(x, w):
    return pl.pallas_call(
        _lin_body,
        out_shape=jax.ShapeDtypeStruct((NP, H), jnp.float32),
    )(x, w)


def _bn_relu(z):
    valid = lax.broadcasted_iota(jnp.int32, (NP, 1), 0) < N
    zm = jnp.where(valid, z, 0.0)
    m = jnp.sum(zm, axis=0, keepdims=True) * jnp.float32(1.0 / N)
    d = z - m
    dv = jnp.where(valid, d * d, 0.0)
    v = jnp.sum(dv, axis=0, keepdims=True) * jnp.float32(1.0 / N)
    return m, d, v


def _post_body(agg_ref, h_ref, b_ref, g_ref, bb_ref, wn_ref, out_ref):
    z = (agg_ref[...] + h_ref[...]) * jnp.float32(C17) + b_ref[...]
    _, d, v = _bn_relu(z)
    y = g_ref[...] * d / jnp.sqrt(v + jnp.float32(1e-5)) + bb_ref[...]
    y = jnp.maximum(y, 0.0)
    out_ref[...] = jnp.dot(y, wn_ref[...],
                           preferred_element_type=jnp.float32, precision=_HP)


def _post(agg, h, b, g, bb, wn):
    return pl.pallas_call(
        _post_body,
        out_shape=jax.ShapeDtypeStruct((NP, H), jnp.float32),
    )(agg, h, b, g, bb, wn)


def _final_body(agg_ref, h_ref, b_ref, g_ref, bb_ref, br_ref, gw_ref, gb_ref,
                aw_ref, ab_ref, w1_ref, b1_ref, w2_ref, b2_ref,
                aff_ref, pose_ref):
    z = (agg_ref[...] + h_ref[...]) * jnp.float32(C17) + b_ref[...]
    _, d, v = _bn_relu(z)
    y = g_ref[...] * d / jnp.sqrt(v + jnp.float32(1e-5)) + bb_ref[...]
    y = jnp.maximum(y, 0.0)

    gate = jnp.dot(y, gw_ref[...], preferred_element_type=jnp.float32,
                   precision=_HP) + gb_ref[...]
    gate = jnp.maximum(gate, 0.0)
    bt = br_ref[...]
    giota = lax.broadcasted_iota(jnp.int32, (1, G), 1)
    msk = bt == giota
    gmax = jnp.max(jnp.where(msk, gate, -jnp.inf), axis=0, keepdims=True)
    gsel = jnp.sum(jnp.where(msk, gmax, 0.0), axis=1, keepdims=True)
    e = jnp.where(msk, jnp.exp(gate - gsel), 0.0)
    s = jnp.sum(e, axis=0, keepdims=True)
    ssafe = jnp.where(s > 0, s, 1.0)
    attn = e / ssafe
    pooled = lax.dot_general(attn, y, (((0,), (0,)), ((), ())),
                             preferred_element_type=jnp.float32,
                             precision=_HP)
    aff_ref[...] = jnp.dot(pooled, aw_ref[...],
                           preferred_element_type=jnp.float32,
                           precision=_HP) + ab_ref[...]
    t = jnp.maximum(jnp.dot(pooled, w1_ref[...],
                            preferred_element_type=jnp.float32,
                            precision=_HP) + b1_ref[...], 0.0)
    pose_ref[...] = jnp.dot(t, w2_ref[...],
                            preferred_element_type=jnp.float32,
                            precision=_HP) + b2_ref[...]


def _final(agg, h, b, g, bb, batch_r, gw, gb, aw, ab, w1, b1, w2, b2):
    return pl.pallas_call(
        _final_body,
        out_shape=(jax.ShapeDtypeStruct((G, 1), jnp.float32),
                   jax.ShapeDtypeStruct((G, 3), jnp.float32)),
    )(agg, h, b, g, bb, batch_r, gw, gb, aw, ab, w1, b1, w2, b2)


@functools.lru_cache(maxsize=1)
def _make_agg():
    mesh = plsc.VectorSubcoreMesh(core_axis_name="c", subcore_axis_name="s")

    nch = NODES_W // CHN
    nbuf = 4

    @functools.partial(
        pl.kernel,
        out_type=jax.ShapeDtypeStruct((NP, H), jnp.float32),
        mesh=mesh,
        scratch_types=[
            pltpu.VMEM((EDG_W,), jnp.int32),
            pltpu.VMEM((nbuf, CHN * K, H), jnp.float32),
            pltpu.VMEM((CHN, H), jnp.float32),
            pltpu.SemaphoreType.DMA((nbuf,)),
        ],
    )
    def agg(h_hbm, src_hbm, out_hbm, idx_v, rows_v, acc_v, sems):
        w = lax.axis_index("s") * 2 + lax.axis_index("c")
        pltpu.sync_copy(src_hbm.at[pl.ds(w * EDG_W, EDG_W)], idx_v)

        def gather(ci, b):
            return pltpu.make_async_copy(
                h_hbm.at[idx_v.at[pl.ds(ci * (CHN * K), CHN * K)]],
                rows_v.at[b], sems.at[b])

        for b in range(nbuf):
            gather(b, b).start()

        @pl.loop(0, nch, step=nbuf)
        def _chunks(ci):
            for b in range(nbuf):
                cur = ci + b
                gather(cur, b).wait()
                for n in range(CHN):
                    for p in range(H // 16):
                        sl = pl.ds(p * 16, 16)
                        acc = rows_v[b, n * K, sl]
                        for k in range(1, K):
                            acc = acc + rows_v[b, n * K + k, sl]
                        acc_v[n, sl] = acc

                @pl.when(cur + nbuf < nch)
                def _():
                    gather(cur + nbuf, b).start()

                pltpu.sync_copy(
                    acc_v, out_hbm.at[pl.ds(w * NODES_W + cur * CHN, CHN)])

    return agg


def kernel(x, pos, batch, conv1_W, conv1_b, bn1_g, bn1_b, conv2_W, conv2_b,
           bn2_g, bn2_b, conv3_W, conv3_b, bn3_g, bn3_b, gate_W, gate_b,
           aff_W, aff_b, rl_W1, rl_b1, rl_W2, rl_b2):
    f32 = jnp.float32
    i32 = jnp.int32

    xin = jnp.concatenate([x, pos], axis=1)
    xin = jnp.pad(xin, ((0, NP - N), (0, 0)))
    pos8 = jnp.pad(pos, ((0, NP - N), (0, 5)))
    posT8 = pos8.T
    sq = jnp.pad(jnp.sum(pos * pos, axis=1), (0, NP - N))
    sq_r = sq.reshape(NP, 1)
    sq_c = sq.reshape(1, NP)
    batch_pad = jnp.concatenate(
        [batch.astype(i32), jnp.full((NP - N,), -1, i32)])
    batch_r = batch_pad.reshape(NP, 1)
    batch_c = batch_pad.reshape(1, NP)

    gids = jnp.arange(G, dtype=i32)
    seg_start = jnp.searchsorted(batch, gids, side="left").astype(i32)
    seg_end = jnp.searchsorted(batch, gids, side="right").astype(i32)
    r0s = jnp.arange(NBLK, dtype=i32) * RB
    g_lo = batch[jnp.minimum(r0s, N - 1)]
    g_hi = batch[jnp.minimum(r0s + RB - 1, N - 1)]
    col_lo = seg_start[g_lo]
    col_hi = seg_end[g_hi]
    clo = (col_lo // CB).astype(i32)
    ccnt = jnp.maximum((col_hi + CB - 1) // CB - clo, 1).astype(i32)

    topi = _knn_topk(pos8, posT8, sq_r, sq_c, batch_r, batch_c,
                     clo, ccnt)
    src_flat = topi.reshape(-1)

    b1 = conv1_b.reshape(1, H)
    b2 = conv2_b.reshape(1, H)
    b3 = conv3_b.reshape(1, H)
    g1, g2, g3 = bn1_g.reshape(1, H), bn2_g.reshape(1, H), bn3_g.reshape(1, H)
    bb1, bb2, bb3 = (bn1_b.reshape(1, H), bn2_b.reshape(1, H),
                     bn3_b.reshape(1, H))

    aggf = _make_agg()
    h1 = _lin(xin, conv1_W)
    a1 = aggf(h1, src_flat)
    h2 = _post(a1, h1, b1, g1, bb1, conv2_W)
    a2 = aggf(h2, src_flat)
    h3 = _post(a2, h2, b2, g2, bb2, conv3_W)
    a3 = aggf(h3, src_flat)

    affinity, pose = _final(
        a3, h3, b3, g3, bb3, batch_r,
        gate_W, gate_b.reshape(1, 1), aff_W, aff_b.reshape(1, 1),
        rl_W1, rl_b1.reshape(1, H), rl_W2, rl_b2.reshape(1, 3))
    return (affinity, pose)


# speedup vs baseline: 1.6569x; 1.6569x over previous
import functools

import numpy as np
import jax
import jax.numpy as jnp
from jax import lax
from jax.experimental import pallas as pl
from jax.experimental.pallas import tpu as pltpu
from jax.experimental.pallas import tpu_sc as plsc

N = 10000
G = 16
K = 16
H = 128
NP = 10240
RB = 256
CB = 512
NBLK = NP // RB
BIG = 1e10
HUGE = 1e30
NW = 32
NODES_W = NP // NW
CHN = 16
EDG_W = NODES_W * K

_rs17 = np.float32(1.0) / np.sqrt(np.float32(17.0))
C17 = float(np.float32(_rs17 * _rs17))
_HP = lax.Precision.HIGHEST


def _knn_body(clo_ref, ccnt_ref, posr_ref, posc_ref, sqr_ref, sqc_ref,
              br_ref, bc_ref, topi_ref):
    i = pl.program_id(0)
    r0 = i * RB
    rbatch = bc_ref[0:1, pl.ds(r0, RB)]
    riota = r0 + lax.broadcasted_iota(jnp.int32, (1, RB), 1)
    prowT = posc_ref[:, pl.ds(r0, RB)]
    sqr = sqc_ref[0:1, pl.ds(r0, RB)]

    topv0 = jnp.full((K, RB), HUGE, jnp.float32)
    topi0 = jnp.zeros((K, RB), jnp.int32)
    cblk0 = clo_ref[i]

    def chunk(t, carry):
        topv, topi = carry
        c = (cblk0 + t) * CB
        dotv = jnp.dot(posr_ref[pl.ds(c, CB), :], prowT,
                       preferred_element_type=jnp.float32)
        d2 = (sqr_ref[pl.ds(c, CB), :] + sqr) - 2.0 * dotv
        cbatch = br_ref[pl.ds(c, CB), :]
        ciota = c + lax.broadcasted_iota(jnp.int32, (CB, 1), 0)
        masked = (cbatch != rbatch) | (ciota == riota)
        d2 = jnp.where(masked, jnp.float32(BIG), d2)

        A = jnp.concatenate([topv, d2], axis=0)
        I = jnp.concatenate([topi, jnp.broadcast_to(ciota, (CB, RB))], axis=0)
        vs, js = [], []
        for _ in range(K):
            m = jnp.min(A, axis=0, keepdims=True)
            sel = A == m
            j = jnp.min(jnp.where(sel, I, jnp.int32(2147483647)),
                        axis=0, keepdims=True)
            vs.append(m)
            js.append(j)
            A = jnp.where(sel & (I == j), jnp.float32(HUGE), A)
        return (jnp.concatenate(vs, axis=0), jnp.concatenate(js, axis=0))

    topv, topi = lax.fori_loop(0, ccnt_ref[i], chunk, (topv0, topi0))
    topi_ref[...] = topi


def _knn_topk(pos8, posT8, sq_r, sq_c, batch_r, batch_c, clo, ccnt):
    grid_spec = pltpu.PrefetchScalarGridSpec(
        num_scalar_prefetch=2,
        grid=(NBLK,),
        in_specs=[
            pl.BlockSpec((NP, 8), lambda i, *_: (0, 0)),
            pl.BlockSpec((8, NP), lambda i, *_: (0, 0)),
            pl.BlockSpec((NP, 1), lambda i, *_: (0, 0)),
            pl.BlockSpec((1, NP), lambda i, *_: (0, 0)),
            pl.BlockSpec((NP, 1), lambda i, *_: (0, 0)),
            pl.BlockSpec((1, NP), lambda i, *_: (0, 0)),
        ],
        out_specs=pl.BlockSpec((K, RB), lambda i, *_: (0, i)),
    )
    return pl.pallas_call(
        _knn_body,
        grid_spec=grid_spec,
        out_shape=jax.ShapeDtypeStruct((K, NP), jnp.int32),
    )(clo, ccnt, pos8, posT8, sq_r, sq_c, batch_r, batch_c)


def _lin_body(x_ref, w_ref, out_ref):
    out_ref[...] = jnp.dot(x_ref[...], w_ref[...],
                           preferred_element_type=jnp.float32, precision=_HP)


def _lin(x, w):
    return pl.pallas_call(
        _lin_body,
        out_shape=jax.ShapeDtypeStruct((NP, H), jnp.float32),
    )(x, w)


def _bn_relu(z):
    valid = lax.broadcasted_iota(jnp.int32, (NP, 1), 0) < N
    zm = jnp.where(valid, z, 0.0)
    m = jnp.sum(zm, axis=0, keepdims=True) * jnp.float32(1.0 / N)
    d = z - m
    dv = jnp.where(valid, d * d, 0.0)
    v = jnp.sum(dv, axis=0, keepdims=True) * jnp.float32(1.0 / N)
    return m, d, v


def _post_body(agg_ref, h_ref, b_ref, g_ref, bb_ref, wn_ref, out_ref):
    z = (agg_ref[...] + h_ref[...]) * jnp.float32(C17) + b_ref[...]
    _, d, v = _bn_relu(z)
    y = g_ref[...] * d / jnp.sqrt(v + jnp.float32(1e-5)) + bb_ref[...]
    y = jnp.maximum(y, 0.0)
    out_ref[...] = jnp.dot(y, wn_ref[...],
                           preferred_element_type=jnp.float32, precision=_HP)


def _post(agg, h, b, g, bb, wn):
    return pl.pallas_call(
        _post_body,
        out_shape=jax.ShapeDtypeStruct((NP, H), jnp.float32),
    )(agg, h, b, g, bb, wn)


def _final_body(agg_ref, h_ref, b_ref, g_ref, bb_ref, br_ref, gw_ref, gb_ref,
                aw_ref, ab_ref, w1_ref, b1_ref, w2_ref, b2_ref,
                aff_ref, pose_ref):
    z = (agg_ref[...] + h_ref[...]) * jnp.float32(C17) + b_ref[...]
    _, d, v = _bn_relu(z)
    y = g_ref[...] * d / jnp.sqrt(v + jnp.float32(1e-5)) + bb_ref[...]
    y = jnp.maximum(y, 0.0)

    gate = jnp.dot(y, gw_ref[...], preferred_element_type=jnp.float32,
                   precision=_HP) + gb_ref[...]
    gate = jnp.maximum(gate, 0.0)
    bt = br_ref[...]
    giota = lax.broadcasted_iota(jnp.int32, (1, G), 1)
    msk = bt == giota
    gmax = jnp.max(jnp.where(msk, gate, -jnp.inf), axis=0, keepdims=True)
    gsel = jnp.sum(jnp.where(msk, gmax, 0.0), axis=1, keepdims=True)
    e = jnp.where(msk, jnp.exp(gate - gsel), 0.0)
    s = jnp.sum(e, axis=0, keepdims=True)
    ssafe = jnp.where(s > 0, s, 1.0)
    attn = e / ssafe
    pooled = lax.dot_general(attn, y, (((0,), (0,)), ((), ())),
                             preferred_element_type=jnp.float32,
                             precision=_HP)
    aff_ref[...] = jnp.dot(pooled, aw_ref[...],
                           preferred_element_type=jnp.float32,
                           precision=_HP) + ab_ref[...]
    t = jnp.maximum(jnp.dot(pooled, w1_ref[...],
                            preferred_element_type=jnp.float32,
                            precision=_HP) + b1_ref[...], 0.0)
    pose_ref[...] = jnp.dot(t, w2_ref[...],
                            preferred_element_type=jnp.float32,
                            precision=_HP) + b2_ref[...]


def _final(agg, h, b, g, bb, batch_r, gw, gb, aw, ab, w1, b1, w2, b2):
    return pl.pallas_call(
        _final_body,
        out_shape=(jax.ShapeDtypeStruct((G, 1), jnp.float32),
                   jax.ShapeDtypeStruct((G, 3), jnp.float32)),
    )(agg, h, b, g, bb, batch_r, gw, gb, aw, ab, w1, b1, w2, b2)


@functools.lru_cache(maxsize=1)
def _make_agg():
    mesh = plsc.VectorSubcoreMesh(core_axis_name="c", subcore_axis_name="s")

    nch = NODES_W // CHN
    nbuf = 2

    @functools.partial(
        pl.kernel,
        out_type=jax.ShapeDtypeStruct((NP, H), jnp.float32),
        mesh=mesh,
        scratch_types=[
            pltpu.VMEM((EDG_W,), jnp.int32),
            pltpu.VMEM((nbuf, CHN * K, H), jnp.float32),
            pltpu.VMEM((CHN, H), jnp.float32),
            pltpu.SemaphoreType.DMA((nbuf,)),
        ],
    )
    def agg(h_hbm, src_hbm, out_hbm, idx_v, rows_v, acc_v, sems):
        w = lax.axis_index("s") * 2 + lax.axis_index("c")
        pltpu.sync_copy(src_hbm.at[pl.ds(w * EDG_W, EDG_W)], idx_v)

        def gather(ci, b):
            return pltpu.make_async_copy(
                h_hbm.at[idx_v.at[pl.ds(ci * (CHN * K), CHN * K)]],
                rows_v.at[b], sems.at[b])

        for b in range(nbuf):
            gather(b, b).start()

        @pl.loop(0, nch, step=nbuf)
        def _chunks(ci):
            for b in range(nbuf):
                cur = ci + b
                gather(cur, b).wait()
                for n in range(CHN):
                    for p in range(H // 16):
                        sl = pl.ds(p * 16, 16)
                        acc = rows_v[b, n * K, sl]
                        for k in range(1, K):
                            acc = acc + rows_v[b, n * K + k, sl]
                        acc_v[n, sl] = acc

                @pl.when(cur + nbuf < nch)
                def _():
                    gather(cur + nbuf, b).start()

                pltpu.sync_copy(
                    acc_v, out_hbm.at[pl.ds(w * NODES_W + cur * CHN, CHN)])

    return agg


def kernel(x, pos, batch, conv1_W, conv1_b, bn1_g, bn1_b, conv2_W, conv2_b,
           bn2_g, bn2_b, conv3_W, conv3_b, bn3_g, bn3_b, gate_W, gate_b,
           aff_W, aff_b, rl_W1, rl_b1, rl_W2, rl_b2):
    f32 = jnp.float32
    i32 = jnp.int32

    xin = jnp.concatenate([x, pos], axis=1)
    xin = jnp.pad(xin, ((0, NP - N), (0, 0)))
    pos8 = jnp.pad(pos, ((0, NP - N), (0, 5)))
    posT8 = pos8.T
    sq = jnp.pad(jnp.sum(pos * pos, axis=1), (0, NP - N))
    sq_r = sq.reshape(NP, 1)
    sq_c = sq.reshape(1, NP)
    batch_pad = jnp.concatenate(
        [batch.astype(i32), jnp.full((NP - N,), -1, i32)])
    batch_r = batch_pad.reshape(NP, 1)
    batch_c = batch_pad.reshape(1, NP)

    gids = jnp.arange(G, dtype=i32)
    seg_start = jnp.searchsorted(batch, gids, side="left").astype(i32)
    seg_end = jnp.searchsorted(batch, gids, side="right").astype(i32)
    r0s = jnp.arange(NBLK, dtype=i32) * RB
    g_lo = batch[jnp.minimum(r0s, N - 1)]
    g_hi = batch[jnp.minimum(r0s + RB - 1, N - 1)]
    col_lo = seg_start[g_lo]
    col_hi = seg_end[g_hi]
    clo = (col_lo // CB).astype(i32)
    ccnt = jnp.maximum((col_hi + CB - 1) // CB - clo, 1).astype(i32)

    topiT = _knn_topk(pos8, posT8, sq_r, sq_c, batch_r, batch_c,
                      clo, ccnt)
    src_flat = topiT.T.reshape(-1)

    b1 = conv1_b.reshape(1, H)
    b2 = conv2_b.reshape(1, H)
    b3 = conv3_b.reshape(1, H)
    g1, g2, g3 = bn1_g.reshape(1, H), bn2_g.reshape(1, H), bn3_g.reshape(1, H)
    bb1, bb2, bb3 = (bn1_b.reshape(1, H), bn2_b.reshape(1, H),
                     bn3_b.reshape(1, H))

    aggf = _make_agg()
    h1 = _lin(xin, conv1_W)
    a1 = aggf(h1, src_flat)
    h2 = _post(a1, h1, b1, g1, bb1, conv2_W)
    a2 = aggf(h2, src_flat)
    h3 = _post(a2, h2, b2, g2, bb2, conv3_W)
    a3 = aggf(h3, src_flat)

    affinity, pose = _final(
        a3, h3, b3, g3, bb3, batch_r,
        gate_W, gate_b.reshape(1, 1), aff_W, aff_b.reshape(1, 1),
        rl_W1, rl_b1.reshape(1, H), rl_W2, rl_b2.reshape(1, 3))
    return (affinity, pose)
